# baseline (speedup 1.0000x reference)
import functools

import jax
import jax.numpy as jnp
from jax import lax
from jax.experimental import pallas as pl
from jax.experimental.pallas import tpu as pltpu
from jax.experimental.pallas import tpu_sc as plsc

_SIDE_NUM = 1000
_VEC_LEN = 32
_B = 16384

_NC = 2
_NS = 16
_NW = _NC * _NS
_BPW = _B // _NW
_CHUNK = 128
_NCHUNK = _BPW // _CHUNK
_QSTEPS = _BPW // 16


def _quantize(v):
    v = v + jnp.float32(50.0)
    v = jnp.maximum(v, jnp.float32(1e-8))
    v = jnp.minimum(v, jnp.float32(100.0))
    p = v * jnp.float32(999.0 / 100.0)
    i0 = p.astype(jnp.int32)
    frac = p - i0.astype(jnp.float32)
    half = jnp.float32(0.5)
    odd = (i0 & 1) == 1
    inc = (frac > half) | ((frac == half) & odd)
    return jnp.where(inc, i0 + 1, i0)


def _body(xs_hbm, ys_hbm, table_hbm, out_hbm, xs_v, ys_v, ids_v, rows_v, sem):
    wid = lax.axis_index("s") * _NC + lax.axis_index("c")
    base = wid * _BPW

    pltpu.sync_copy(xs_hbm.at[pl.ds(base, _BPW)], xs_v)
    pltpu.sync_copy(ys_hbm.at[pl.ds(base, _BPW)], ys_v)

    copies = []
    for c in range(_NCHUNK):
        for jj in range(_CHUNK // 16):
            j = c * (_CHUNK // 16) + jj
            x = xs_v[pl.ds(j * 16, 16)]
            y = ys_v[pl.ds(j * 16, 16)]
            ids_v[pl.ds(j * 16, 16)] = _quantize(x) * _SIDE_NUM + _quantize(y)
        copies.append(pltpu.async_copy(
            table_hbm.at[ids_v.at[pl.ds(c * _CHUNK, _CHUNK)]],
            rows_v.at[pl.ds(c * _CHUNK, _CHUNK)],
            sem))
    for cp in copies:
        cp.wait()

    pltpu.sync_copy(rows_v, out_hbm.at[pl.ds(base, _BPW)])


@jax.jit
def kernel(locations, pos_emb):
    xs = locations[:, 0]
    ys = locations[:, 1]
    mesh = plsc.VectorSubcoreMesh(core_axis_name="c", subcore_axis_name="s")
    run = pl.kernel(
        _body,
        mesh=mesh,
        out_type=jax.ShapeDtypeStruct((_B, _VEC_LEN), jnp.float32),
        scratch_types=[
            pltpu.VMEM((_BPW,), jnp.float32),
            pltpu.VMEM((_BPW,), jnp.float32),
            pltpu.VMEM((_BPW,), jnp.int32),
            pltpu.VMEM((_BPW, _VEC_LEN), jnp.float32),
            pltpu.SemaphoreType.DMA,
        ],
        compiler_params=pltpu.CompilerParams(use_tc_tiling_on_sc=False),
    )
    return run(xs, ys, pos_emb)
